# baseline (speedup 1.0000x reference)
import functools

import jax
import jax.numpy as jnp
from jax import lax
from jax.experimental import pallas as pl
from jax.experimental.pallas import tpu as pltpu
from jax.experimental.pallas import tpu_sc as plsc

NC = 2
NS = 16
L = 16
NW = NC * NS
ROWS = 128
COLS = 32768
ROWS_PER_W = ROWS // NW
NBLK = COLS // L
U = 8
NIT = NBLK // U
BIG = 2**31 - 1


def _row_argmax(buf):
    iota = lax.iota(jnp.int32, L)
    init_max = tuple(jnp.full((L,), -jnp.inf, jnp.float32) for _ in range(U))
    init_blk = tuple(jnp.zeros((L,), jnp.int32) for _ in range(U))

    def step(i, carry):
        maxs, blks = carry
        base = i * (U * L)
        new_maxs = []
        new_blks = []
        for u in range(U):
            chunk = buf[pl.ds(base + u * L, L)]
            m = chunk > maxs[u]
            new_maxs.append(jnp.where(m, chunk, maxs[u]))
            new_blks.append(jnp.where(m, i, blks[u]))
        return tuple(new_maxs), tuple(new_blks)

    maxs, blks = lax.fori_loop(0, NIT, step, (init_max, init_blk))

    vmax = maxs[0]
    vpos = blks[0] * (U * L) + iota
    for u in range(1, U):
        pu = blks[u] * (U * L) + (u * L) + iota
        better = (maxs[u] > vmax) | ((maxs[u] == vmax) & (pu < vpos))
        vmax = jnp.where(better, maxs[u], vmax)
        vpos = jnp.where(better, pu, vpos)

    gmax = jnp.max(vmax)
    cand = jnp.where(vmax == gmax, vpos, BIG)
    return jnp.min(cand)


def _sc_body(in_hbm, out_hbm, buf0, buf1, res_buf, sem0, sem1):
    wid = lax.axis_index("s") * NC + lax.axis_index("c")
    row0 = wid * ROWS_PER_W

    bufs = (buf0, buf1)
    sems = (sem0, sem1)
    pltpu.make_async_copy(in_hbm.at[row0], buf0, sem0).start()

    res = jnp.zeros((L,), jnp.int32)
    iota = lax.iota(jnp.int32, L)
    for r in range(ROWS_PER_W):
        buf = bufs[r % 2]
        sem = sems[r % 2]
        if r + 1 < ROWS_PER_W:
            pltpu.make_async_copy(
                in_hbm.at[row0 + r + 1], bufs[(r + 1) % 2], sems[(r + 1) % 2]
            ).start()
        pltpu.make_async_copy(in_hbm.at[row0 + r], buf, sem).wait()
        idx = _row_argmax(buf)
        res = jnp.where(iota == r, idx, res)

    res_buf[...] = res
    pltpu.sync_copy(res_buf, out_hbm.at[wid])


@jax.jit
def kernel(input):
    mesh = plsc.VectorSubcoreMesh(core_axis_name="c", subcore_axis_name="s")
    cp = pltpu.CompilerParams(needs_layout_passes=False)
    sc = pl.kernel(
        _sc_body,
        out_type=jax.ShapeDtypeStruct((NW, L), jnp.int32),
        mesh=mesh,
        scratch_types=[
            pltpu.VMEM((COLS,), jnp.float32),
            pltpu.VMEM((COLS,), jnp.float32),
            pltpu.VMEM((L,), jnp.int32),
            pltpu.SemaphoreType.DMA,
            pltpu.SemaphoreType.DMA,
        ],
        compiler_params=cp,
    )
    packed = sc(input)
    return packed[:, :ROWS_PER_W].reshape(ROWS)
